# baseline (speedup 1.0000x reference)
import functools

import jax
import jax.numpy as jnp
from jax import lax
from jax.experimental import pallas as pl
from jax.experimental.pallas import tpu as pltpu
from jax.experimental.pallas import tpu_sc as plsc

D = 8192
H = 8192
DT = D // 2

_NW = 32
_L = 16


_GCHUNK = DT // _NW


def _sc_gather(x, tidx):
    mesh = plsc.VectorSubcoreMesh(core_axis_name="c", subcore_axis_name="s")

    @functools.partial(
        pl.kernel,
        out_type=jax.ShapeDtypeStruct((DT,), jnp.float32),
        mesh=mesh,
        scratch_types=[
            pltpu.VMEM((D,), jnp.float32),
            pltpu.VMEM((_GCHUNK,), jnp.int32),
            pltpu.VMEM((_GCHUNK,), jnp.float32),
        ],
        compiler_params=pltpu.CompilerParams(needs_layout_passes=False),
    )
    def k(x_hbm, tidx_hbm, out_hbm, x_v, idx_v, out_v):
        wid = lax.axis_index("s") * 2 + lax.axis_index("c")
        base = wid * _GCHUNK
        pltpu.sync_copy(x_hbm, x_v)
        pltpu.sync_copy(tidx_hbm.at[pl.ds(base, _GCHUNK)], idx_v)
        for i in range(_GCHUNK // _L):
            idx = idx_v[pl.ds(i * _L, _L)]
            out_v[pl.ds(i * _L, _L)] = plsc.load_gather(x_v, [idx])
        pltpu.sync_copy(out_v, out_hbm.at[pl.ds(base, _GCHUNK)])

    return k(x, tidx)


_WIN = D // _NW


def _sc_scatter(x, yt, midx, tidx):
    mesh = plsc.VectorSubcoreMesh(core_axis_name="c", subcore_axis_name="s")

    @functools.partial(
        pl.kernel,
        out_type=jax.ShapeDtypeStruct((D,), jnp.float32),
        mesh=mesh,
        scratch_types=[
            pltpu.VMEM((DT,), jnp.float32),
            pltpu.VMEM((DT,), jnp.int32),
            pltpu.VMEM((_WIN,), jnp.float32),
        ],
        compiler_params=pltpu.CompilerParams(needs_layout_passes=False),
    )
    def k(x_hbm, yt_hbm, midx_hbm, tidx_hbm, y_hbm, yt_v, tidx_v, y_v):
        wid = lax.axis_index("s") * 2 + lax.axis_index("c")
        lo = wid * _WIN
        hi = lo + _WIN
        pltpu.sync_copy(x_hbm.at[pl.ds(lo, _WIN)], y_v)
        pltpu.sync_copy(yt_hbm, yt_v)
        pltpu.sync_copy(tidx_hbm, tidx_v)

        def tr_body(i, carry):
            ti = tidx_v[pl.ds(i * _L, _L)]
            vals = yt_v[pl.ds(i * _L, _L)]
            inw = (ti >= lo) & (ti < hi)
            plsc.store_scatter(y_v, [ti - lo], vals, mask=inw)
            return carry

        lax.fori_loop(0, DT // _L, tr_body, 0)
        pltpu.sync_copy(y_v, y_hbm.at[pl.ds(lo, _WIN)])

    return k(x, yt, midx, tidx)


_T1 = 32
_R1 = DT // _T1
_BN2 = 256
_T2 = H // _BN2


def _tc_body(xt_ref, w1_ref, b1_ref, w2_ref, b2_ref, xtf_ref, yt_ref,
             ld_ref, h_ref, acc_ref, acch_ref):
    i = pl.program_id(0)

    @pl.when(i < _T1)
    def _stage1():
        part = jnp.dot(xt_ref[...], w1_ref[...],
                       preferred_element_type=jnp.float32)

        @pl.when(i == 0)
        def _init_h():
            acch_ref[...] = part

        @pl.when(i > 0)
        def _acc_h():
            acch_ref[...] = acch_ref[...] + part

    @pl.when(i == _T1)
    def _activate():
        hh = jnp.tanh(acch_ref[...] + b1_ref[...])
        for r in range(_T2):
            h_ref[r, :] = hh[0, r * _BN2:(r + 1) * _BN2]

    @pl.when(i >= _T1)
    def _stage2():
        j = i - _T1
        hr = h_ref[pl.ds(j, 1), :]
        part = jnp.dot(hr, w2_ref[...], preferred_element_type=jnp.float32)

        @pl.when(i == _T1)
        def _init():
            acc_ref[...] = part

        @pl.when(i > _T1)
        def _acc():
            acc_ref[...] = acc_ref[...] + part

        @pl.when(i == _T1 + _T2 - 1)
        def _final():
            params = acc_ref[...] + b2_ref[...]
            s = params[:, :DT]
            t = params[:, DT:]
            yt_ref[...] = xtf_ref[...] * jnp.exp(s) + t
            ld_ref[0, 0] = jnp.sum(s)


def _tc_mlp(xt, W1, b1, W2, b2):
    xt2 = xt.reshape(1, DT)
    b1r = b1.reshape(1, H)
    b2r = b2.reshape(1, 2 * DT)
    j2 = lambda i: jnp.clip(i - _T1, 0, _T2 - 1)
    k1 = lambda i: jnp.minimum(i, _T1 - 1)
    yt, ld = pl.pallas_call(
        _tc_body,
        grid=(_T1 + _T2,),
        in_specs=[
            pl.BlockSpec((1, _R1), lambda i: (0, k1(i))),
            pl.BlockSpec((_R1, H), lambda i: (k1(i), 0)),
            pl.BlockSpec((1, H), lambda i: (0, 0)),
            pl.BlockSpec((_BN2, 2 * DT), lambda i: (j2(i), 0)),
            pl.BlockSpec((1, 2 * DT), lambda i: (0, 0)),
            pl.BlockSpec((1, DT), lambda i: (0, 0)),
        ],
        out_specs=[
            pl.BlockSpec((1, DT), lambda i: (0, 0)),
            pl.BlockSpec((1, 1), lambda i: (0, 0),
                         memory_space=pltpu.SMEM),
        ],
        out_shape=[
            jax.ShapeDtypeStruct((1, DT), jnp.float32),
            jax.ShapeDtypeStruct((1, 1), jnp.float32),
        ],
        scratch_shapes=[
            pltpu.VMEM((_T2, _BN2), jnp.float32),
            pltpu.VMEM((1, 2 * DT), jnp.float32),
            pltpu.VMEM((1, H), jnp.float32),
        ],
        compiler_params=pltpu.CompilerParams(
            dimension_semantics=("arbitrary",)),
    )(xt2, W1, b1r, W2, b2r, xt2)
    return yt.reshape(DT), ld[0, 0]


def kernel(x, W1, b1, W2, b2, mask_idx, transform_idx):
    x_t = _sc_gather(x, transform_idx)
    y_t, log_det = _tc_mlp(x_t, W1, b1, W2, b2)
    y = _sc_scatter(x, y_t, mask_idx, transform_idx)
    return (y, log_det)


# speedup vs baseline: 1.0504x; 1.0504x over previous
import functools

import jax
import jax.numpy as jnp
from jax import lax
from jax.experimental import pallas as pl
from jax.experimental.pallas import tpu as pltpu
from jax.experimental.pallas import tpu_sc as plsc

D = 8192
H = 8192
DT = D // 2

_NW = 32
_L = 16


_GCHUNK = DT // _NW


def _sc_gather(x, tidx):
    mesh = plsc.VectorSubcoreMesh(core_axis_name="c", subcore_axis_name="s")

    @functools.partial(
        pl.kernel,
        out_type=jax.ShapeDtypeStruct((DT,), jnp.float32),
        mesh=mesh,
        scratch_types=[
            pltpu.VMEM((D,), jnp.float32),
            pltpu.VMEM((_GCHUNK,), jnp.int32),
            pltpu.VMEM((_GCHUNK,), jnp.float32),
        ],
        compiler_params=pltpu.CompilerParams(needs_layout_passes=False),
    )
    def k(x_hbm, tidx_hbm, out_hbm, x_v, idx_v, out_v):
        wid = lax.axis_index("s") * 2 + lax.axis_index("c")
        base = wid * _GCHUNK
        pltpu.sync_copy(x_hbm, x_v)
        pltpu.sync_copy(tidx_hbm.at[pl.ds(base, _GCHUNK)], idx_v)
        for i in range(_GCHUNK // _L):
            idx = idx_v[pl.ds(i * _L, _L)]
            out_v[pl.ds(i * _L, _L)] = plsc.load_gather(x_v, [idx])
        pltpu.sync_copy(out_v, out_hbm.at[pl.ds(base, _GCHUNK)])

    return k(x, tidx)


_WIN = D // _NW


def _sc_scatter(x, yt, midx, tidx):
    mesh = plsc.VectorSubcoreMesh(core_axis_name="c", subcore_axis_name="s")

    @functools.partial(
        pl.kernel,
        out_type=jax.ShapeDtypeStruct((D,), jnp.float32),
        mesh=mesh,
        scratch_types=[
            pltpu.VMEM((DT,), jnp.float32),
            pltpu.VMEM((DT,), jnp.int32),
            pltpu.VMEM((_WIN,), jnp.float32),
        ],
        compiler_params=pltpu.CompilerParams(needs_layout_passes=False),
    )
    def k(x_hbm, yt_hbm, midx_hbm, tidx_hbm, y_hbm, yt_v, tidx_v, y_v):
        wid = lax.axis_index("s") * 2 + lax.axis_index("c")
        lo = wid * _WIN
        hi = lo + _WIN
        pltpu.sync_copy(x_hbm.at[pl.ds(lo, _WIN)], y_v)
        pltpu.sync_copy(yt_hbm, yt_v)
        pltpu.sync_copy(tidx_hbm, tidx_v)

        def tr_body(i, carry):
            ti = tidx_v[pl.ds(i * _L, _L)]
            vals = yt_v[pl.ds(i * _L, _L)]
            inw = (ti >= lo) & (ti < hi)
            plsc.store_scatter(y_v, [ti - lo], vals, mask=inw)
            return carry

        lax.fori_loop(0, DT // _L, tr_body, 0)
        pltpu.sync_copy(y_v, y_hbm.at[pl.ds(lo, _WIN)])

    return k(x, yt, midx, tidx)


_T1 = 16
_R1 = DT // _T1
_BN2 = 256
_T2 = H // _BN2


def _tc_body(xt_ref, w1_ref, b1_ref, w2_ref, b2_ref, xtf_ref, yt_ref,
             ld_ref, h_ref, acc_ref, acch_ref):
    i = pl.program_id(0)

    @pl.when(i < _T1)
    def _stage1():
        part = jnp.dot(xt_ref[...], w1_ref[...],
                       preferred_element_type=jnp.float32)

        @pl.when(i == 0)
        def _init_h():
            acch_ref[...] = part

        @pl.when(i > 0)
        def _acc_h():
            acch_ref[...] = acch_ref[...] + part

    @pl.when(i == _T1)
    def _activate():
        hh = jnp.tanh(acch_ref[...] + b1_ref[...])
        for r in range(_T2):
            h_ref[r, :] = hh[0, r * _BN2:(r + 1) * _BN2]

    @pl.when(i >= _T1)
    def _stage2():
        j = i - _T1
        hr = h_ref[pl.ds(j, 1), :]
        part = jnp.dot(hr, w2_ref[...], preferred_element_type=jnp.float32)

        @pl.when(i == _T1)
        def _init():
            acc_ref[...] = part

        @pl.when(i > _T1)
        def _acc():
            acc_ref[...] = acc_ref[...] + part

        @pl.when(i == _T1 + _T2 - 1)
        def _final():
            params = acc_ref[...] + b2_ref[...]
            s = params[:, :DT]
            t = params[:, DT:]
            yt_ref[...] = xtf_ref[...] * jnp.exp(s) + t
            ld_ref[0, 0] = jnp.sum(s)


def _tc_mlp(xt, W1, b1, W2, b2):
    xt2 = xt.reshape(1, DT)
    b1r = b1.reshape(1, H)
    b2r = b2.reshape(1, 2 * DT)
    j2 = lambda i: jnp.clip(i - _T1, 0, _T2 - 1)
    k1 = lambda i: jnp.minimum(i, _T1 - 1)
    yt, ld = pl.pallas_call(
        _tc_body,
        grid=(_T1 + _T2,),
        in_specs=[
            pl.BlockSpec((1, _R1), lambda i: (0, k1(i))),
            pl.BlockSpec((_R1, H), lambda i: (k1(i), 0)),
            pl.BlockSpec((1, H), lambda i: (0, 0)),
            pl.BlockSpec((_BN2, 2 * DT), lambda i: (j2(i), 0)),
            pl.BlockSpec((1, 2 * DT), lambda i: (0, 0)),
            pl.BlockSpec((1, DT), lambda i: (0, 0)),
        ],
        out_specs=[
            pl.BlockSpec((1, DT), lambda i: (0, 0)),
            pl.BlockSpec((1, 1), lambda i: (0, 0),
                         memory_space=pltpu.SMEM),
        ],
        out_shape=[
            jax.ShapeDtypeStruct((1, DT), jnp.float32),
            jax.ShapeDtypeStruct((1, 1), jnp.float32),
        ],
        scratch_shapes=[
            pltpu.VMEM((_T2, _BN2), jnp.float32),
            pltpu.VMEM((1, 2 * DT), jnp.float32),
            pltpu.VMEM((1, H), jnp.float32),
        ],
        compiler_params=pltpu.CompilerParams(
            dimension_semantics=("arbitrary",)),
    )(xt2, W1, b1r, W2, b2r, xt2)
    return yt.reshape(DT), ld[0, 0]


def kernel(x, W1, b1, W2, b2, mask_idx, transform_idx):
    x_t = _sc_gather(x, transform_idx)
    y_t, log_det = _tc_mlp(x_t, W1, b1, W2, b2)
    y = _sc_scatter(x, y_t, mask_idx, transform_idx)
    return (y, log_det)
